# Initial kernel scaffold; baseline (speedup 1.0000x reference)
import functools

import jax
import jax.numpy as jnp
from jax import lax
from jax.experimental import pallas as pl
from jax.experimental.pallas import tpu as pltpu
from jax.experimental.pallas import tpu_sc as plsc

NUM_EMB = 1000000
DIM = 64
BATCH = 16384
SEQ = 50
B = BATCH * SEQ
NC = 2
NS = 16
NW = NC * NS
BPW = B // NW
SUB = 128
CHUNK = 512
NSUB = CHUNK // SUB
NCHUNK = BPW // CHUNK
IROWS_PER_W = BPW // SUB


def _build():
    mesh = plsc.VectorSubcoreMesh(core_axis_name="c", subcore_axis_name="s")

    @functools.partial(
        pl.kernel,
        mesh=mesh,
        out_type=jax.ShapeDtypeStruct((B, DIM), jnp.float32),
        scratch_types=[
            pltpu.VMEM((NSUB, SUB), jnp.int32),
            pltpu.VMEM((CHUNK, DIM), jnp.float32),
            pltpu.SemaphoreType.DMA,
        ],
    )
    def gather_kernel(ids_hbm, table_hbm, out_hbm, idx_v, rows_v, sem):
        wid = lax.axis_index("s") * NC + lax.axis_index("c")
        base = wid * BPW
        ibase = wid * IROWS_PER_W

        def body(j, carry):
            pltpu.sync_copy(ids_hbm.at[pl.ds(ibase + j * NSUB, NSUB)], idx_v)
            copies = [
                pltpu.async_copy(
                    table_hbm.at[idx_v.at[k]],
                    rows_v.at[pl.ds(k * SUB, SUB)],
                    sem,
                )
                for k in range(NSUB)
            ]
            for cp in copies:
                cp.wait()
            pltpu.sync_copy(rows_v, out_hbm.at[pl.ds(base + j * CHUNK, CHUNK)])
            return carry

        lax.fori_loop(0, NCHUNK, body, 0)

    return gather_kernel


_gather = _build()


def kernel(token_ids, EmbeddingLayer):
    ids = token_ids.astype(jnp.int32).reshape(B // SUB, SUB)
    out = _gather(ids, EmbeddingLayer)
    return out.reshape(BATCH, SEQ, DIM)


# speedup vs baseline: 1.7954x; 1.7954x over previous
import functools

import jax
import jax.numpy as jnp
from jax import lax
from jax.experimental import pallas as pl
from jax.experimental.pallas import tpu as pltpu
from jax.experimental.pallas import tpu_sc as plsc

NUM_EMB = 1000000
DIM = 64
BATCH = 16384
SEQ = 50
B = BATCH * SEQ
NC = 2
NS = 16
NW = NC * NS
BPW = B // NW
SUB = 128
CHUNK = 512
NSUB = CHUNK // SUB
NCHUNK = BPW // CHUNK
IROWS_PER_W = BPW // SUB


def _build():
    mesh = plsc.VectorSubcoreMesh(core_axis_name="c", subcore_axis_name="s")

    @functools.partial(
        pl.kernel,
        mesh=mesh,
        out_type=jax.ShapeDtypeStruct((B, DIM), jnp.float32),
        scratch_types=[
            pltpu.VMEM((NSUB, SUB), jnp.int32),
            pltpu.VMEM((CHUNK, DIM), jnp.float32),
            pltpu.SemaphoreType.DMA,
        ],
        compiler_params=pltpu.CompilerParams(use_tc_tiling_on_sc=False),
    )
    def gather_kernel(ids_hbm, table_hbm, out_hbm, idx_v, rows_v, sem):
        wid = lax.axis_index("s") * NC + lax.axis_index("c")
        base = wid * BPW
        ibase = wid * IROWS_PER_W

        def body(j, carry):
            pltpu.sync_copy(ids_hbm.at[pl.ds(ibase + j * NSUB, NSUB)], idx_v)
            copies = [
                pltpu.async_copy(
                    table_hbm.at[idx_v.at[k]],
                    rows_v.at[pl.ds(k * SUB, SUB)],
                    sem,
                )
                for k in range(NSUB)
            ]
            for cp in copies:
                cp.wait()
            pltpu.sync_copy(rows_v, out_hbm.at[pl.ds(base + j * CHUNK, CHUNK)])
            return carry

        lax.fori_loop(0, NCHUNK, body, 0)

    return gather_kernel


_gather = _build()


def kernel(token_ids, EmbeddingLayer):
    ids = token_ids.astype(jnp.int32).reshape(B // SUB, SUB)
    out = _gather(ids, EmbeddingLayer)
    return out.reshape(BATCH, SEQ, DIM)


# speedup vs baseline: 1.8764x; 1.0451x over previous
import functools

import jax
import jax.numpy as jnp
from jax import lax
from jax.experimental import pallas as pl
from jax.experimental.pallas import tpu as pltpu
from jax.experimental.pallas import tpu_sc as plsc

NUM_EMB = 1000000
DIM = 64
BATCH = 16384
SEQ = 50
B = BATCH * SEQ
NC = 2
NS = 16
NW = NC * NS
BPW = B // NW
SUB = 128
CHUNK = 512
NSUB = CHUNK // SUB
NCHUNK = BPW // CHUNK
IROWS = BPW // SUB
NBUF = 2
G = NCHUNK // NBUF


def _build():
    mesh = plsc.VectorSubcoreMesh(core_axis_name="c", subcore_axis_name="s")

    @functools.partial(
        pl.kernel,
        mesh=mesh,
        out_type=jax.ShapeDtypeStruct((B, DIM), jnp.float32),
        scratch_types=[
            pltpu.VMEM((IROWS, SUB), jnp.int32),
            [pltpu.VMEM((CHUNK, DIM), jnp.float32) for _ in range(NBUF)],
            [pltpu.SemaphoreType.DMA for _ in range(NBUF)],
            [pltpu.SemaphoreType.DMA for _ in range(NBUF)],
        ],
        compiler_params=pltpu.CompilerParams(use_tc_tiling_on_sc=False),
    )
    def gather_kernel(ids_hbm, table_hbm, out_hbm, idx_v, bufs, fsems, wsems):
        wid = lax.axis_index("s") * NC + lax.axis_index("c")
        base = wid * BPW

        pltpu.sync_copy(ids_hbm.at[pl.ds(wid * IROWS, IROWS)], idx_v)

        def fill(chunk, b):
            for k in range(NSUB):
                pltpu.async_copy(
                    table_hbm.at[idx_v.at[chunk * NSUB + k]],
                    bufs[b].at[pl.ds(k * SUB, SUB)],
                    fsems[b],
                )

        def wait_fill(b):
            pltpu.make_async_copy(
                table_hbm.at[pl.ds(0, CHUNK)], bufs[b], fsems[b]
            ).wait()

        def drain(chunk, b):
            pltpu.async_copy(
                bufs[b], out_hbm.at[pl.ds(base + chunk * CHUNK, CHUNK)], wsems[b]
            )

        def wait_drain(b):
            pltpu.make_async_copy(
                bufs[b], out_hbm.at[pl.ds(0, CHUNK)], wsems[b]
            ).wait()

        for b in range(NBUF):
            fill(b, b)

        def outer(g, carry):
            for b in range(NBUF):
                chunk = g * NBUF + b
                wait_fill(b)
                drain(chunk, b)
                wait_drain(b)
                fill(chunk + NBUF, b)
            return carry

        lax.fori_loop(0, G - 1, outer, 0)

        for b in range(NBUF):
            chunk = (G - 1) * NBUF + b
            wait_fill(b)
            drain(chunk, b)
        for b in range(NBUF):
            wait_drain(b)

    return gather_kernel


_gather = _build()


def kernel(token_ids, EmbeddingLayer):
    ids = token_ids.astype(jnp.int32).reshape(B // SUB, SUB)
    out = _gather(ids, EmbeddingLayer)
    return out.reshape(BATCH, SEQ, DIM)
